# baseline (speedup 1.0000x reference)
import jax
import jax.numpy as jnp
from jax import lax
from jax.experimental import pallas as pl
from jax.experimental.pallas import tpu as pltpu
from jax.experimental.pallas import tpu_sc as plsc

N_USER = 5000
N_ITEM = 5000
N_NODES = N_USER + N_ITEM
N_EDGES = 320000
D = 128
ODE_STEPS = 2

NW = 32
CHUNK = 128
CHUNKS_PER_TILE = 79
EDGES_PAD = NW * CHUNKS_PER_TILE * CHUNK
CHUNKS_PER_BLOCK = 2 * CHUNKS_PER_TILE
SLOW_CID = 0
CHUNKS_SLOW = CHUNKS_PER_TILE
CHUNKS_FAST = CHUNKS_PER_BLOCK - CHUNKS_SLOW
NPAD = 10240
TILES_PER_SC = 16
ROWS_PER_TILE = NPAD // TILES_PER_SC


def _mesh():
  return plsc.VectorSubcoreMesh(core_axis_name="c", subcore_axis_name="s",
                                num_cores=2, num_subcores=TILES_PER_SC)


def _sc_deg_body(dst_hbm, znd_hbm, ones_hbm, deg_out, idx_d, ones_v, deg_sh):
  cid = lax.axis_index("c")
  sid = lax.axis_index("s")
  wid = sid * 2 + cid
  r0 = pl.multiple_of(sid * ROWS_PER_TILE, 8)

  pltpu.sync_copy(dst_hbm.at[wid], idx_d)
  pltpu.sync_copy(ones_hbm, ones_v)
  pltpu.sync_copy(znd_hbm.at[pl.ds(r0, ROWS_PER_TILE)],
                  deg_sh.at[pl.ds(r0, ROWS_PER_TILE)])
  plsc.subcore_barrier()

  def loop_body(j, carry):
    pltpu.sync_copy(ones_v, deg_sh.at[idx_d.at[j]], add=True)
    return carry

  lax.fori_loop(0, CHUNKS_PER_TILE, loop_body, 0)
  plsc.subcore_barrier()
  pltpu.sync_copy(deg_sh.at[pl.ds(r0, ROWS_PER_TILE)],
                  deg_out.at[cid, pl.ds(r0, ROWS_PER_TILE)])


def _sc_step_body(adj_hbm, x_hbm, znd_hbm, acc_out,
                  idx, rows, acc_sh, gsem, isem):
  cid = lax.axis_index("c")
  sid = lax.axis_index("s")
  wid = sid * 2 + cid
  r0 = pl.multiple_of(sid * ROWS_PER_TILE, 8)

  pltpu.sync_copy(znd_hbm.at[pl.ds(r0, ROWS_PER_TILE)],
                  acc_sh.at[pl.ds(r0, ROWS_PER_TILE)])
  plsc.subcore_barrier()

  pltpu.sync_copy(adj_hbm.at[wid, 0], idx.at[0])
  pltpu.async_copy(x_hbm.at[idx.at[0, pl.ds(0, CHUNK)]], rows.at[0], gsem)
  pltpu.async_copy(adj_hbm.at[wid, 1], idx.at[1], isem)

  def loop_body(j, carry):
    slot2 = lax.rem(j, 2)
    nxt2 = lax.rem(j + 1, 2)
    nxt3 = lax.rem(j + 1, 3)
    pre3 = lax.rem(j + 2, 3)

    @pl.when(j + 1 < CHUNKS_PER_TILE)
    def _():
      pltpu.make_async_copy(adj_hbm.at[wid, 0], idx.at[0], isem).wait()
      pltpu.async_copy(x_hbm.at[idx.at[nxt3, pl.ds(0, CHUNK)]],
                       rows.at[nxt2], gsem)

      @pl.when(j + 2 < CHUNKS_PER_TILE)
      def _():
        pltpu.async_copy(adj_hbm.at[wid, j + 2], idx.at[pre3], isem)

    pltpu.make_async_copy(x_hbm.at[idx.at[0, pl.ds(0, CHUNK)]],
                          rows.at[0], gsem).wait()
    pltpu.sync_copy(
        rows.at[slot2],
        acc_sh.at[idx.at[lax.rem(j, 3), pl.ds(CHUNK, CHUNK)]], add=True)
    return carry

  lax.fori_loop(0, CHUNKS_PER_TILE, loop_body, 0)
  plsc.subcore_barrier()

  pltpu.sync_copy(acc_sh.at[pl.ds(r0, ROWS_PER_TILE)],
                  acc_out.at[cid, pl.ds(r0, ROWS_PER_TILE)])


_SC_CACHE = {}


def _sc_deg():
  if "deg" not in _SC_CACHE:
    _SC_CACHE["deg"] = pl.kernel(
        _sc_deg_body,
        out_type=[jax.ShapeDtypeStruct((2, NPAD, D), jnp.float32)],
        mesh=_mesh(),
        scratch_types=[
            pltpu.VMEM((CHUNKS_PER_TILE, CHUNK), jnp.int32),
            pltpu.VMEM((CHUNK, D), jnp.float32),
            pltpu.VMEM_SHARED((NPAD, D), jnp.float32),
        ])
  return _SC_CACHE["deg"]


def _sc_step():
  if "step" not in _SC_CACHE:
    _SC_CACHE["step"] = pl.kernel(
        _sc_step_body,
        out_type=[jax.ShapeDtypeStruct((2, NPAD, D), jnp.float32)],
        mesh=_mesh(),
        scratch_types=[
            pltpu.VMEM((3, 2 * CHUNK), jnp.int32),
            pltpu.VMEM((2, CHUNK, D), jnp.float32),
            pltpu.VMEM_SHARED((NPAD, D), jnp.float32),
            pltpu.SemaphoreType.DMA,
            pltpu.SemaphoreType.DMA,
        ])
  return _SC_CACHE["step"]


def _tc_update1(x_ref, e_ref, acc_ref, deg_ref, h_ref, out_ref, msq_ref):
  x = x_ref[...]
  h = h_ref[0, 0]
  deg = deg_ref[0, :, 0:1] + deg_ref[1, :, 0:1]
  inv = 1.0 / jnp.maximum(deg, 1.0)
  agg = (acc_ref[0] + acc_ref[1]) * inv
  out_ref[...] = x + h * (agg - x + e_ref[...])
  msq_ref[0, 0] = jnp.max(jnp.sum(x * x, axis=1))


def _tc_update2(x_ref, e_ref, acc_ref, deg_ref, h_ref, msq_ref,
                out_u_ref, out_i_ref):
  x = x_ref[...]
  h = h_ref[0, 0]
  scale = 1.0 / jnp.sqrt(msq_ref[0, 0])
  deg = deg_ref[0, :, 0:1] + deg_ref[1, :, 0:1]
  inv = 1.0 / jnp.maximum(deg, 1.0)
  agg = (acc_ref[0] + acc_ref[1]) * inv
  res = (x + h * (agg - x + e_ref[...])) * scale
  out_u_ref[...] = res[:N_USER]
  out_i_ref[...] = res[N_USER:N_NODES]


def _tc1(x, e, acc, deg, h):
  return pl.pallas_call(
      _tc_update1,
      out_shape=(jax.ShapeDtypeStruct((NPAD, D), jnp.float32),
                 jax.ShapeDtypeStruct((1, 1), jnp.float32)),
      in_specs=[pl.BlockSpec(memory_space=pltpu.VMEM)] * 4 +
               [pl.BlockSpec(memory_space=pltpu.SMEM)],
      out_specs=(pl.BlockSpec(memory_space=pltpu.VMEM),
                 pl.BlockSpec(memory_space=pltpu.SMEM)),
  )(x, e, acc, deg, h)


def _tc2(x, e, acc, deg, h, msq):
  return pl.pallas_call(
      _tc_update2,
      out_shape=(jax.ShapeDtypeStruct((N_USER, D), jnp.float32),
                 jax.ShapeDtypeStruct((N_ITEM, D), jnp.float32)),
      in_specs=[pl.BlockSpec(memory_space=pltpu.VMEM)] * 4 +
               [pl.BlockSpec(memory_space=pltpu.SMEM)] * 2,
      out_specs=(pl.BlockSpec(memory_space=pltpu.VMEM),
                 pl.BlockSpec(memory_space=pltpu.VMEM)),
  )(x, e, acc, deg, h, msq)


@jax.jit
def kernel(adj_obs, t_diff, x_u, x_i, xu_embed, xi_embed):
  x0 = jnp.concatenate([x_u, x_i], axis=0)
  e = jnp.concatenate([xu_embed, xi_embed], axis=0)
  pad = jnp.zeros((NPAD - N_NODES, D), jnp.float32)
  x0p = jnp.concatenate([x0, pad], axis=0)
  ep = jnp.concatenate([e, pad], axis=0)

  npe = EDGES_PAD - N_EDGES
  srcp = jnp.concatenate([adj_obs[0], jnp.zeros((npe,), jnp.int32)])
  dstp = jnp.concatenate([adj_obs[1], jnp.full((npe,), N_NODES, jnp.int32)])
  src3 = srcp.reshape(NW, CHUNKS_PER_TILE, CHUNK)
  dst3 = dstp.reshape(NW, CHUNKS_PER_TILE, CHUNK)
  adj3 = jnp.concatenate([src3[:, :, None, :], dst3[:, :, None, :]],
                         axis=2).reshape(NW, CHUNKS_PER_TILE, 2 * CHUNK)

  zeros_nd = jnp.zeros((NPAD, D), jnp.float32)
  ones_cd = jnp.ones((CHUNK, D), jnp.float32)
  h = (t_diff / ODE_STEPS).reshape(1, 1)

  (deg,) = _sc_deg()(dst3, zeros_nd, ones_cd)
  (acc_a,) = _sc_step()(adj3, x0p, zeros_nd)
  x1p, msq = _tc1(x0p, ep, acc_a, deg, h)
  (acc_b,) = _sc_step()(adj3, x1p, zeros_nd)
  return _tc2(x1p, ep, acc_b, deg, h, msq)


# speedup vs baseline: 1.0892x; 1.0892x over previous
import jax
import jax.numpy as jnp
from jax import lax
from jax.experimental import pallas as pl
from jax.experimental.pallas import tpu as pltpu
from jax.experimental.pallas import tpu_sc as plsc

N_USER = 5000
N_ITEM = 5000
N_NODES = N_USER + N_ITEM
N_EDGES = 320000
D = 128
ODE_STEPS = 2

NW = 32
CHUNK = 128
CHUNKS_PER_TILE = 79
EDGES_PAD = NW * CHUNKS_PER_TILE * CHUNK
NPAD = 10240
TILES_PER_SC = 16
ROWS_PER_TILE = NPAD // TILES_PER_SC


def _mesh():
  return plsc.VectorSubcoreMesh(core_axis_name="c", subcore_axis_name="s",
                                num_cores=2, num_subcores=TILES_PER_SC)


def _sc_deg_body(dst_hbm, znd_hbm, ones_hbm, deg_out, idx_d, ones_v, deg_sh):
  cid = lax.axis_index("c")
  sid = lax.axis_index("s")
  wid = sid * 2 + cid
  r0 = pl.multiple_of(sid * ROWS_PER_TILE, 8)

  pltpu.sync_copy(dst_hbm.at[wid], idx_d)
  pltpu.sync_copy(ones_hbm, ones_v)
  pltpu.sync_copy(znd_hbm.at[pl.ds(r0, ROWS_PER_TILE)],
                  deg_sh.at[pl.ds(r0, ROWS_PER_TILE)])
  plsc.subcore_barrier()

  def loop_body(j, carry):
    pltpu.sync_copy(ones_v, deg_sh.at[idx_d.at[j]], add=True)
    return carry

  lax.fori_loop(0, CHUNKS_PER_TILE, loop_body, 0)
  plsc.subcore_barrier()
  pltpu.sync_copy(deg_sh.at[pl.ds(r0, ROWS_PER_TILE)],
                  deg_out.at[cid, pl.ds(r0, ROWS_PER_TILE)])


def _sc_step_body(src_hbm, dst_hbm, x_hbm, znd_hbm, acc_out,
                  idx_s, idx_d, rows, acc_sh, gsem, isem):
  cid = lax.axis_index("c")
  sid = lax.axis_index("s")
  wid = sid * 2 + cid
  r0 = pl.multiple_of(sid * ROWS_PER_TILE, 8)

  pltpu.sync_copy(znd_hbm.at[pl.ds(r0, ROWS_PER_TILE)],
                  acc_sh.at[pl.ds(r0, ROWS_PER_TILE)])
  plsc.subcore_barrier()

  pltpu.sync_copy(src_hbm.at[wid, 0], idx_s.at[0])
  pltpu.sync_copy(dst_hbm.at[wid, 0], idx_d.at[0])
  pltpu.async_copy(x_hbm.at[idx_s.at[0]], rows.at[0], gsem)
  pltpu.async_copy(src_hbm.at[wid, 1], idx_s.at[1], isem)
  pltpu.async_copy(dst_hbm.at[wid, 1], idx_d.at[1], isem)

  def loop_body(j, carry):
    slot2 = lax.rem(j, 2)
    nxt2 = lax.rem(j + 1, 2)
    nxt3 = lax.rem(j + 1, 3)
    pre3 = lax.rem(j + 2, 3)

    @pl.when(j + 1 < CHUNKS_PER_TILE)
    def _():
      pltpu.make_async_copy(src_hbm.at[wid, 0], idx_s.at[0], isem).wait()
      pltpu.make_async_copy(dst_hbm.at[wid, 0], idx_d.at[0], isem).wait()
      pltpu.async_copy(x_hbm.at[idx_s.at[nxt3]], rows.at[nxt2], gsem)

      @pl.when(j + 2 < CHUNKS_PER_TILE)
      def _():
        pltpu.async_copy(src_hbm.at[wid, j + 2], idx_s.at[pre3], isem)
        pltpu.async_copy(dst_hbm.at[wid, j + 2], idx_d.at[pre3], isem)

    pltpu.make_async_copy(x_hbm.at[idx_s.at[0]], rows.at[0], gsem).wait()
    pltpu.sync_copy(rows.at[slot2], acc_sh.at[idx_d.at[lax.rem(j, 3)]],
                    add=True)
    return carry

  lax.fori_loop(0, CHUNKS_PER_TILE, loop_body, 0)
  plsc.subcore_barrier()

  pltpu.sync_copy(acc_sh.at[pl.ds(r0, ROWS_PER_TILE)],
                  acc_out.at[cid, pl.ds(r0, ROWS_PER_TILE)])


_SC_CACHE = {}


def _sc_deg():
  if "deg" not in _SC_CACHE:
    _SC_CACHE["deg"] = pl.kernel(
        _sc_deg_body,
        out_type=[jax.ShapeDtypeStruct((2, NPAD, D), jnp.float32)],
        mesh=_mesh(),
        scratch_types=[
            pltpu.VMEM((CHUNKS_PER_TILE, CHUNK), jnp.int32),
            pltpu.VMEM((CHUNK, D), jnp.float32),
            pltpu.VMEM_SHARED((NPAD, D), jnp.float32),
        ])
  return _SC_CACHE["deg"]


def _sc_step():
  if "step" not in _SC_CACHE:
    _SC_CACHE["step"] = pl.kernel(
        _sc_step_body,
        out_type=[jax.ShapeDtypeStruct((2, NPAD, D), jnp.float32)],
        mesh=_mesh(),
        scratch_types=[
            pltpu.VMEM((3, CHUNK), jnp.int32),
            pltpu.VMEM((3, CHUNK), jnp.int32),
            pltpu.VMEM((2, CHUNK, D), jnp.float32),
            pltpu.VMEM_SHARED((NPAD, D), jnp.float32),
            pltpu.SemaphoreType.DMA,
            pltpu.SemaphoreType.DMA,
        ])
  return _SC_CACHE["step"]


def _tc_update1(x_ref, e_ref, acc_ref, deg_ref, h_ref, out_ref, msq_ref):
  x = x_ref[...]
  h = h_ref[0, 0]
  deg = deg_ref[0, :, 0:1] + deg_ref[1, :, 0:1]
  inv = 1.0 / jnp.maximum(deg, 1.0)
  agg = (acc_ref[0] + acc_ref[1]) * inv
  out_ref[...] = x + h * (agg - x + e_ref[...])
  msq_ref[0, 0] = jnp.max(jnp.sum(x * x, axis=1))


def _tc_update2(x_ref, e_ref, acc_ref, deg_ref, h_ref, msq_ref,
                out_u_ref, out_i_ref):
  x = x_ref[...]
  h = h_ref[0, 0]
  scale = 1.0 / jnp.sqrt(msq_ref[0, 0])
  deg = deg_ref[0, :, 0:1] + deg_ref[1, :, 0:1]
  inv = 1.0 / jnp.maximum(deg, 1.0)
  agg = (acc_ref[0] + acc_ref[1]) * inv
  res = (x + h * (agg - x + e_ref[...])) * scale
  out_u_ref[...] = res[:N_USER]
  out_i_ref[...] = res[N_USER:N_NODES]


def _tc1(x, e, acc, deg, h):
  return pl.pallas_call(
      _tc_update1,
      out_shape=(jax.ShapeDtypeStruct((NPAD, D), jnp.float32),
                 jax.ShapeDtypeStruct((1, 1), jnp.float32)),
      in_specs=[pl.BlockSpec(memory_space=pltpu.VMEM)] * 4 +
               [pl.BlockSpec(memory_space=pltpu.SMEM)],
      out_specs=(pl.BlockSpec(memory_space=pltpu.VMEM),
                 pl.BlockSpec(memory_space=pltpu.SMEM)),
  )(x, e, acc, deg, h)


def _tc2(x, e, acc, deg, h, msq):
  return pl.pallas_call(
      _tc_update2,
      out_shape=(jax.ShapeDtypeStruct((N_USER, D), jnp.float32),
                 jax.ShapeDtypeStruct((N_ITEM, D), jnp.float32)),
      in_specs=[pl.BlockSpec(memory_space=pltpu.VMEM)] * 4 +
               [pl.BlockSpec(memory_space=pltpu.SMEM)] * 2,
      out_specs=(pl.BlockSpec(memory_space=pltpu.VMEM),
                 pl.BlockSpec(memory_space=pltpu.VMEM)),
  )(x, e, acc, deg, h, msq)


@jax.jit
def kernel(adj_obs, t_diff, x_u, x_i, xu_embed, xi_embed):
  x0 = jnp.concatenate([x_u, x_i], axis=0)
  e = jnp.concatenate([xu_embed, xi_embed], axis=0)
  pad = jnp.zeros((NPAD - N_NODES, D), jnp.float32)
  x0p = jnp.concatenate([x0, pad], axis=0)
  ep = jnp.concatenate([e, pad], axis=0)

  npe = EDGES_PAD - N_EDGES
  srcp = jnp.concatenate([adj_obs[0], jnp.zeros((npe,), jnp.int32)])
  dstp = jnp.concatenate([adj_obs[1], jnp.full((npe,), N_NODES, jnp.int32)])
  src3 = srcp.reshape(NW, CHUNKS_PER_TILE, CHUNK)
  dst3 = dstp.reshape(NW, CHUNKS_PER_TILE, CHUNK)

  zeros_nd = jnp.zeros((NPAD, D), jnp.float32)
  ones_cd = jnp.ones((CHUNK, D), jnp.float32)
  h = (t_diff / ODE_STEPS).reshape(1, 1)

  (deg,) = _sc_deg()(dst3, zeros_nd, ones_cd)
  (acc_a,) = _sc_step()(src3, dst3, x0p, zeros_nd)
  x1p, msq = _tc1(x0p, ep, acc_a, deg, h)
  (acc_b,) = _sc_step()(src3, dst3, x1p, zeros_nd)
  return _tc2(x1p, ep, acc_b, deg, h, msq)
